# baseline (speedup 1.0000x reference)
import functools

import jax
import jax.numpy as jnp
from jax import lax
from jax.experimental import layout as jex_layout
from jax.experimental import pallas as pl
from jax.experimental.pallas import tpu as pltpu
from jax.experimental.pallas import tpu_sc as plsc

_D = 64
_B = 16384
_H = 50

_info = plsc.get_sparse_core_info()
_NC = _info.num_cores
_NS = _info.num_subcores
_NW = _NC * _NS
_BW = _B // _NW
_SUB = 256
_NSUB = _BW // _SUB
_NBT = _SUB // 128
_NDT = _D // 8


def _make_gather():
    mesh = plsc.VectorSubcoreMesh(core_axis_name="c", subcore_axis_name="s")

    @functools.partial(
        pl.kernel,
        mesh=mesh,
        compiler_params=pltpu.CompilerParams(
            use_tc_tiling_on_sc=False, needs_layout_passes=False),
        out_type=jax.ShapeDtypeStruct((_H, _NDT, _B // 128, 1024), jnp.float32),
        scratch_types=[
            pltpu.VMEM((_H, _BW), jnp.int32),
        ]
        + [pltpu.VMEM((_SUB, _D), jnp.float32) for _ in range(2)]
        + [pltpu.VMEM((_D * _SUB,), jnp.float32) for _ in range(2)]
        + [pltpu.SemaphoreType.DMA for _ in range(4)],
    )
    def gather_kernel(table_hbm, idxt_hbm, out_hbm, idx_v,
                      r0, r1, t0, t1, sg0, sg1, so0, so1):
        rows = (r0, r1)
        tbuf = (t0, t1)
        sg = (sg0, sg1)
        so = (so0, so1)
        wid = lax.axis_index("s") * _NC + lax.axis_index("c")
        b0 = wid * _BW
        pltpu.sync_copy(idxt_hbm.at[:, pl.ds(b0, _BW)], idx_v)
        iota = lax.iota(jnp.int32, 16)
        dmod = [(iota + k) % 16 for k in range(16)]
        dpart = [(m >> 3) * (_NBT * 1024) + (m & 7) * 128 for m in dmod]

        def g_desc(h, sub):
            return pltpu.make_async_copy(
                table_hbm.at[idx_v.at[h, pl.ds(sub * _SUB, _SUB)]],
                rows[sub], sg[sub])

        def o_desc(h, sub, dt, bt):
            return pltpu.make_async_copy(
                tbuf[sub].at[pl.ds((dt * _NBT + bt) * 1024, 1024)],
                out_hbm.at[h, dt, wid * (_BW // 128) + sub * _NBT + bt],
                so[sub])

        g_desc(0, 0).start()

        def round_(r, carry):
            for sub in range(_NSUB):
                g_desc(r, sub).wait()
                if sub == 0:
                    g_desc(r, 1).start()
                else:
                    @pl.when(r < _H - 1)
                    def _():
                        g_desc(r + 1, 0).start()

                @pl.when(r >= 1)
                def _():
                    for dt in range(_NDT):
                        for bt in range(_NBT):
                            o_desc(r - 1, sub, dt, bt).wait()

                @plsc.parallel_loop(0, _SUB // 16, step=1, unroll=2)
                def transpose_bb(bb):
                    grow = iota + bb * 16
                    bpart = (grow >> 7) * 1024 + (grow & 127)
                    for k in range(16):
                        tmpk = dpart[k] + bpart
                        for dd in range(4):
                            gcol = dmod[k] + dd * 16
                            vals = plsc.load_gather(rows[sub], [grow, gcol])
                            plsc.store_scatter(
                                tbuf[sub], [tmpk + dd * (2 * _NBT * 1024)],
                                vals)

                for dt in range(_NDT):
                    for bt in range(_NBT):
                        o_desc(r, sub, dt, bt).start()
            return carry

        lax.fori_loop(0, _H, round_, 0)
        for sub in range(_NSUB):
            for dt in range(_NDT):
                for bt in range(_NBT):
                    o_desc(_H - 1, sub, dt, bt).wait()

    return gather_kernel


_gather = _make_gather()


def kernel(inputs, table):
    idxt = inputs.astype(jnp.int32).T
    out = _gather(table, idxt)
    y = out.reshape(_H, _NDT, _B // 128, 8, 128)
    res = y.transpose(2, 4, 0, 1, 3).reshape(_B, _H, _D)
    return jex_layout.with_layout_constraint(
        res, jex_layout.Layout(major_to_minor=(1, 2, 0)))


# speedup vs baseline: 1.4232x; 1.4232x over previous
import functools

import jax
import jax.numpy as jnp
from jax import lax
from jax.experimental import layout as jex_layout
from jax.experimental import pallas as pl
from jax.experimental.pallas import tpu as pltpu
from jax.experimental.pallas import tpu_sc as plsc

_D = 64
_B = 16384
_H = 50

_info = plsc.get_sparse_core_info()
_NC = _info.num_cores
_NS = _info.num_subcores
_NW = _NC * _NS
_BW = _B // _NW
_SUB = 256
_NSUB = _BW // _SUB
_NBT = _SUB // 128
_NDT = _D // 8


def _make_gather():
    mesh = plsc.VectorSubcoreMesh(core_axis_name="c", subcore_axis_name="s")

    @functools.partial(
        pl.kernel,
        mesh=mesh,
        compiler_params=pltpu.CompilerParams(
            use_tc_tiling_on_sc=False, needs_layout_passes=False),
        out_type=jax.ShapeDtypeStruct((_H, _NDT, _B // 128, 1024), jnp.float32),
        scratch_types=[
            pltpu.VMEM((_H, _BW), jnp.int32),
        ]
        + [pltpu.VMEM((_SUB, _D), jnp.float32) for _ in range(2)]
        + [pltpu.VMEM((_D * _SUB,), jnp.float32) for _ in range(2)]
        + [pltpu.SemaphoreType.DMA for _ in range(4)],
    )
    def gather_kernel(table_hbm, idxt_hbm, out_hbm, idx_v,
                      r0, r1, t0, t1, sg0, sg1, so0, so1):
        rows = (r0, r1)
        tbuf = (t0, t1)
        sg = (sg0, sg1)
        so = (so0, so1)
        wid = lax.axis_index("s") * _NC + lax.axis_index("c")
        b0 = wid * _BW
        pltpu.sync_copy(idxt_hbm.at[:, pl.ds(b0, _BW)], idx_v)
        iota = lax.iota(jnp.int32, 16)
        dmod = [(iota + k) % 16 for k in range(16)]
        dpart = [(m >> 3) * (_NBT * 1024) + (m & 7) * 128 for m in dmod]

        def g_desc(h, sub):
            return pltpu.make_async_copy(
                table_hbm.at[idx_v.at[h, pl.ds(sub * _SUB, _SUB)]],
                rows[sub], sg[sub])

        def o_desc(h, sub, dt, bt):
            return pltpu.make_async_copy(
                tbuf[sub].at[pl.ds((dt * _NBT + bt) * 1024, 1024)],
                out_hbm.at[h, dt, wid * (_BW // 128) + sub * _NBT + bt],
                so[sub])

        g_desc(0, 0).start()

        def round_(r, carry):
            for sub in range(_NSUB):
                g_desc(r, sub).wait()
                if sub == 0:
                    g_desc(r, 1).start()
                else:
                    @pl.when(r < _H - 1)
                    def _():
                        g_desc(r + 1, 0).start()

                @pl.when(r >= 1)
                def _():
                    for dt in range(_NDT):
                        for bt in range(_NBT):
                            o_desc(r - 1, sub, dt, bt).wait()

                @plsc.parallel_loop(0, _SUB // 16, step=1, unroll=8)
                def transpose_bb(bb):
                    grow = iota + bb * 16
                    bpart = (grow >> 7) * 1024 + (grow & 127)
                    for k in range(16):
                        tmpk = dpart[k] + bpart
                        for dd in range(4):
                            gcol = dmod[k] + dd * 16
                            vals = plsc.load_gather(rows[sub], [grow, gcol])
                            plsc.store_scatter(
                                tbuf[sub], [tmpk + dd * (2 * _NBT * 1024)],
                                vals)

                for dt in range(_NDT):
                    for bt in range(_NBT):
                        o_desc(r, sub, dt, bt).start()
            return carry

        lax.fori_loop(0, _H, round_, 0)
        for sub in range(_NSUB):
            for dt in range(_NDT):
                for bt in range(_NBT):
                    o_desc(_H - 1, sub, dt, bt).wait()

    return gather_kernel


_gather = _make_gather()


def kernel(inputs, table):
    idxt = inputs.astype(jnp.int32).T
    out = _gather(table, idxt)
    y = out.reshape(_H, _NDT, _B // 128, 8, 128)
    res = y.transpose(2, 4, 0, 1, 3).reshape(_B, _H, _D)
    return jex_layout.with_layout_constraint(
        res, jex_layout.Layout(major_to_minor=(1, 2, 0)))


# speedup vs baseline: 2.4301x; 1.7075x over previous
import functools

import jax
import jax.numpy as jnp
from jax import lax
from jax.experimental import layout as jex_layout
from jax.experimental import pallas as pl
from jax.experimental.pallas import tpu as pltpu
from jax.experimental.pallas import tpu_sc as plsc

_D = 64
_B = 16384
_H = 50

_info = plsc.get_sparse_core_info()
_NC = _info.num_cores
_NS = _info.num_subcores
_NW = _NC * _NS
_BW = _B // _NW
_SUB = 256
_NSUB = _BW // _SUB
_NBT = _SUB // 128
_NDT = _D // 8


def _make_gather():
    mesh = plsc.VectorSubcoreMesh(core_axis_name="c", subcore_axis_name="s")

    @functools.partial(
        pl.kernel,
        mesh=mesh,
        compiler_params=pltpu.CompilerParams(
            use_tc_tiling_on_sc=False, needs_layout_passes=False),
        out_type=jax.ShapeDtypeStruct((_H, _NDT, _B // 128, 1024), jnp.float32),
        scratch_types=[
            pltpu.VMEM((_H, _BW), jnp.int32),
        ]
        + [pltpu.VMEM((_SUB, _D), jnp.float32) for _ in range(2)]
        + [pltpu.VMEM((_D * _SUB,), jnp.float32) for _ in range(2)]
        + [pltpu.SemaphoreType.DMA for _ in range(4)],
    )
    def gather_kernel(table_hbm, idxt_hbm, out_hbm, idx_v,
                      r0, r1, t0, t1, sg0, sg1, so0, so1):
        rows = (r0, r1)
        tbuf = (t0, t1)
        sg = (sg0, sg1)
        so = (so0, so1)
        wid = lax.axis_index("s") * _NC + lax.axis_index("c")
        b0 = wid * _BW
        pltpu.sync_copy(idxt_hbm.at[:, pl.ds(b0, _BW)], idx_v)
        iota = lax.iota(jnp.int32, 16)
        dmod = [(iota + k) % 16 for k in range(16)]
        dpart = [(m >> 3) * (_NBT * 1024) + (m & 7) * 128 for m in dmod]

        def g_desc(h, sub):
            return pltpu.make_async_copy(
                table_hbm.at[idx_v.at[h, pl.ds(sub * _SUB, _SUB)]],
                rows[sub], sg[sub])

        def o_desc(h, sub, dt, bt):
            return pltpu.make_async_copy(
                tbuf[sub].at[pl.ds((dt * _NBT + bt) * 1024, 1024)],
                out_hbm.at[h, dt, wid * (_BW // 128) + sub * _NBT + bt],
                so[sub])

        g_desc(0, 0).start()

        def round_(r, carry):
            for sub in range(_NSUB):
                g_desc(r, sub).wait()
                if sub == 0:
                    g_desc(r, 1).start()
                else:
                    @pl.when(r < _H - 1)
                    def _():
                        g_desc(r + 1, 0).start()

                @pl.when(r >= 1)
                def _():
                    for dt in range(_NDT):
                        for bt in range(_NBT):
                            o_desc(r - 1, sub, dt, bt).wait()

                @plsc.parallel_loop(0, _SUB // 16, step=1, unroll=4)
                def transpose_bb(bb):
                    grow = iota + bb * 16
                    bpart = (grow >> 7) * 1024 + (grow & 127)
                    for k in range(16):
                        tmpk = dpart[k] + bpart
                        for dd in range(4):
                            gcol = dmod[k] + dd * 16
                            vals = plsc.load_gather(rows[sub], [grow, gcol])
                            plsc.store_scatter(
                                tbuf[sub], [tmpk + dd * (2 * _NBT * 1024)],
                                vals)

                for dt in range(_NDT):
                    for bt in range(_NBT):
                        o_desc(r, sub, dt, bt).start()
            return carry

        lax.fori_loop(0, _H, round_, 0)
        for sub in range(_NSUB):
            for dt in range(_NDT):
                for bt in range(_NBT):
                    o_desc(_H - 1, sub, dt, bt).wait()

    return gather_kernel


_gather = _make_gather()


def kernel(inputs, table):
    idxt = inputs.astype(jnp.int32).T
    out = _gather(table, idxt)
    y = out.reshape(_H, _NDT, _B // 128, 8, 128)
    res = y.transpose(2, 4, 0, 1, 3).reshape(_B, _H, _D)
    return jex_layout.with_layout_constraint(
        res, jex_layout.Layout(major_to_minor=(1, 2, 0)))
